# baseline (speedup 1.0000x reference)
import functools
import math

import jax
import jax.numpy as jnp
from jax import lax
from jax.experimental import pallas as pl
from jax.experimental.pallas import tpu as pltpu
from jax.experimental.pallas import tpu_sc as plsc

_N = 4096
_B = 128
_H = 128
_NF = 128
_G = 50
_GP = 64
_L = 6
_CE = 32
_CUTOFF = 10.0
_K = 32
_E = _N * _K

_STEP = _CUTOFF / (_G - 1)
_COEFF = -0.5 / (_STEP * _STEP)
_LOG2 = math.log(2.0)

_ROWS = 128
_TN = 128
_TE = _TN * _K

_NC = 2
_NS = 16
_NW = _NC * _NS
_EPW = _E // _NW
_CH = 128
_NCH = _EPW // _CH


def _ssp(x):
    m = jnp.maximum(x, 0.0)
    return m + jnp.log(jnp.exp(x - m) + jnp.exp(-m)) - _LOG2


_W = 768


def _topk_scan(dist, mask, colid, idx_ref, ew_ref, c_ref):
    rank = jnp.where(mask, dist, 1e9)
    for k in range(_K):
        m = jnp.min(rank, axis=1, keepdims=True)
        amin = jnp.min(jnp.where(rank == m, colid, _N), axis=1, keepdims=True)
        valid = m < _CUTOFF
        idx_ref[:, k:k + 1] = amin
        ew_ref[:, k:k + 1] = jnp.where(valid, m, _CUTOFF)
        c_ref[:, k:k + 1] = jnp.where(
            valid, 0.5 * (jnp.cos(m * (math.pi / _CUTOFF)) + 1.0), 0.0)
        rank = jnp.where(colid == amin, 1e9, rank)


def _graph_body(posr_ref, posc_ref, batr_ref, batc_ref, idx_ref, ew_ref, c_ref):
    t = pl.program_id(0)
    pr = posr_ref[...]
    sqr = jnp.sum(pr * pr, axis=1, keepdims=True)
    br = batr_ref[...]
    bc = batc_ref[...]
    col1 = lax.broadcasted_iota(jnp.int32, (1, _N), 1)
    b_lo = jnp.min(br)
    b_hi = jnp.max(br)
    c_lo = jnp.min(jnp.where(bc == b_lo, col1, _N))
    c_hi = jnp.max(jnp.where(bc == b_hi, col1, 0))
    start = (jnp.minimum(c_lo, _N - _W) // 128) * 128
    start = pl.multiple_of(start, 128)
    fits = (c_hi - start) < _W

    @pl.when(fits)
    def _fast():
        pcw = posc_ref[:, pl.ds(start, _W)]
        sqcw = jnp.sum(pcw * pcw, axis=0, keepdims=True)
        d2 = sqr + sqcw - 2.0 * jnp.dot(pr, pcw,
                                        preferred_element_type=jnp.float32)
        dist = jnp.sqrt(jnp.maximum(d2, 1e-12))
        colw = lax.broadcasted_iota(jnp.int32, (_ROWS, _W), 1) + start
        rowid = lax.broadcasted_iota(jnp.int32, (_ROWS, _W), 0) + t * _ROWS
        bcw = batc_ref[:, pl.ds(start, _W)]
        mask = (br == bcw) & (rowid != colw) & (dist < _CUTOFF)
        _topk_scan(dist, mask, colw, idx_ref, ew_ref, c_ref)

    @pl.when(jnp.logical_not(fits))
    def _slow():
        pc = posc_ref[...]
        sqc = jnp.sum(pc * pc, axis=0, keepdims=True)
        d2 = sqr + sqc - 2.0 * jnp.dot(pr, pc,
                                       preferred_element_type=jnp.float32)
        dist = jnp.sqrt(jnp.maximum(d2, 1e-12))
        colid = lax.broadcasted_iota(jnp.int32, (_ROWS, _N), 1)
        rowid = lax.broadcasted_iota(jnp.int32, (_ROWS, _N), 0) + t * _ROWS
        mask = (br == bc) & (rowid != colid) & (dist < _CUTOFF)
        _topk_scan(dist, mask, colid, idx_ref, ew_ref, c_ref)


def _build_graph(pos_pad, posT, batr, batc):
    return pl.pallas_call(
        _graph_body,
        grid=(_N // _ROWS,),
        in_specs=[
            pl.BlockSpec((_ROWS, 8), lambda i: (i, 0)),
            pl.BlockSpec((8, _N), lambda i: (0, 0)),
            pl.BlockSpec((_ROWS, 1), lambda i: (i, 0)),
            pl.BlockSpec((1, _N), lambda i: (0, 0)),
        ],
        out_specs=[
            pl.BlockSpec((_ROWS, _K), lambda i: (i, 0)),
            pl.BlockSpec((_ROWS, _K), lambda i: (i, 0)),
            pl.BlockSpec((_ROWS, _K), lambda i: (i, 0)),
        ],
        out_shape=[
            jax.ShapeDtypeStruct((_N, _K), jnp.int32),
            jax.ShapeDtypeStruct((_N, _K), jnp.float32),
            jax.ShapeDtypeStruct((_N, _K), jnp.float32),
        ],
    )(pos_pad, posT, batr, batc)


def _embed_body(z_ref, emb_ref, l1_ref, h_ref, x1_ref):
    oh = (z_ref[...] == lax.broadcasted_iota(jnp.int32, (1, 128), 1))
    h = jnp.dot(oh.astype(jnp.float32), emb_ref[...],
                preferred_element_type=jnp.float32)
    h_ref[...] = h
    x1_ref[...] = jnp.dot(h, l1_ref[...], preferred_element_type=jnp.float32)


def _embed(zcol, emb_pad, lin1_0):
    return pl.pallas_call(
        _embed_body,
        grid=(_N // 256,),
        in_specs=[
            pl.BlockSpec((256, 1), lambda i: (i, 0)),
            pl.BlockSpec((128, _H), lambda i: (0, 0)),
            pl.BlockSpec((_H, _H), lambda i: (0, 0)),
        ],
        out_specs=[
            pl.BlockSpec((256, _H), lambda i: (i, 0)),
            pl.BlockSpec((256, _H), lambda i: (i, 0)),
        ],
        out_shape=[
            jax.ShapeDtypeStruct((_N, _H), jnp.float32),
            jax.ShapeDtypeStruct((_N, _H), jnp.float32),
        ],
    )(zcol, emb_pad, lin1_0)


_CHG = 256
_NG = _EPW // _CHG


def _gather_sc_body(x1_hbm, src_hbm, out_hbm, idx_v, bufa, bufb, xs,
                    sga, sgb, swa, swb):
    if True:
        c = lax.axis_index("c")
        s = lax.axis_index("s")
        wid = s * _NC + c
        base = pl.multiple_of(wid * _EPW, 8)
        rs = pl.multiple_of(s * (_N // _NS), 8)
        pltpu.sync_copy(x1_hbm.at[pl.ds(rs, _N // _NS)],
                        xs.at[pl.ds(rs, _N // _NS)])
        pltpu.sync_copy(src_hbm.at[pl.ds(base, _EPW)], idx_v)
        plsc.subcore_barrier()

        def startg(buf, sem, g):
            o0 = pl.multiple_of(g * _CHG, 8)
            o1 = pl.multiple_of(g * _CHG + _CH, 8)
            pltpu.async_copy(xs.at[idx_v.at[pl.ds(o0, _CH)]],
                             buf.at[pl.ds(0, _CH)], sem)
            pltpu.async_copy(xs.at[idx_v.at[pl.ds(o1, _CH)]],
                             buf.at[pl.ds(_CH, _CH)], sem)

        def waitg(buf, sem):
            pltpu.make_async_copy(xs.at[pl.ds(0, _CHG)], buf, sem).wait()

        def startw(buf, sem, g):
            o = pl.multiple_of(base + g * _CHG, 8)
            pltpu.async_copy(buf, out_hbm.at[pl.ds(o, _CHG)], sem)

        def waitw(buf, sem):
            pltpu.make_async_copy(buf, out_hbm.at[pl.ds(0, _CHG)], sem).wait()

        startg(bufa, sga, 0)

        def body(tt, carry):
            g0 = tt * 2

            @pl.when(tt > 0)
            def _():
                waitw(bufb, swb)
            startg(bufb, sgb, g0 + 1)
            waitg(bufa, sga)
            startw(bufa, swa, g0)

            @pl.when(tt + 1 < _NG // 2)
            def _():
                waitw(bufa, swa)
                startg(bufa, sga, g0 + 2)
            waitg(bufb, sgb)
            startw(bufb, swb, g0 + 1)
            return carry

        lax.fori_loop(0, _NG // 2, body, 0)
        waitw(bufa, swa)
        waitw(bufb, swb)


@functools.lru_cache(maxsize=1)
def _gather_kernel():
    return pl.kernel(
        _gather_sc_body,
        out_type=jax.ShapeDtypeStruct((_E, _H), jnp.float32),
        mesh=plsc.VectorSubcoreMesh(core_axis_name="c", subcore_axis_name="s"),
        scratch_types=[
            pltpu.VMEM((_EPW,), jnp.int32),
            pltpu.VMEM((_CHG, _H), jnp.float32),
            pltpu.VMEM((_CHG, _H), jnp.float32),
            pltpu.VMEM_SHARED((_N, _H), jnp.float32),
            pltpu.SemaphoreType.DMA,
            pltpu.SemaphoreType.DMA,
            pltpu.SemaphoreType.DMA,
            pltpu.SemaphoreType.DMA,
        ],
    )


def _gather_rows(x1, src):
    return _gather_kernel()(x1, src)


def _layer_body(gat_ref, ew_ref, c_ref, h_ref, w0_ref, b0_ref, w2_ref, b2_ref,
                l2w_ref, l2b_ref, lw_ref, lb_ref, l1n_ref, hout_ref, x1out_ref):
    ew = ew_ref[...]
    off = lax.broadcasted_iota(jnp.int32, (1, _GP), 1).astype(jnp.float32) * _STEP
    ea = jnp.exp(_COEFF * (ew - off) ** 2)
    f = _ssp(jnp.dot(ea, w0_ref[...], preferred_element_type=jnp.float32)
             + b0_ref[...])
    wf = jnp.dot(f, w2_ref[...], preferred_element_type=jnp.float32) + b2_ref[...]
    wf = wf * c_ref[...]
    msg = gat_ref[...] * wf
    agg = jnp.sum(msg.reshape(_TN, _K, _H), axis=1)
    x2 = _ssp(jnp.dot(agg, l2w_ref[...], preferred_element_type=jnp.float32)
              + l2b_ref[...])
    x2 = jnp.dot(x2, lw_ref[...], preferred_element_type=jnp.float32) + lb_ref[...]
    h = h_ref[...] + x2
    hout_ref[...] = h
    x1out_ref[...] = jnp.dot(h, l1n_ref[...], preferred_element_type=jnp.float32)


def _layer(gat, ew2, c2, h, w0, b0, w2, b2, l2w, l2b, lw, lb, l1n):
    full = lambda a, b: pl.BlockSpec((a, b), lambda i: (0, 0))
    return pl.pallas_call(
        _layer_body,
        grid=(_N // _TN,),
        in_specs=[
            pl.BlockSpec((_TE, _H), lambda i: (i, 0)),
            pl.BlockSpec((_TE, 1), lambda i: (i, 0)),
            pl.BlockSpec((_TE, 1), lambda i: (i, 0)),
            pl.BlockSpec((_TN, _H), lambda i: (i, 0)),
            full(_GP, _NF), full(1, _NF), full(_NF, _NF), full(1, _NF),
            full(_NF, _H), full(1, _H), full(_H, _H), full(1, _H),
            full(_H, _H),
        ],
        out_specs=[
            pl.BlockSpec((_TN, _H), lambda i: (i, 0)),
            pl.BlockSpec((_TN, _H), lambda i: (i, 0)),
        ],
        out_shape=[
            jax.ShapeDtypeStruct((_N, _H), jnp.float32),
            jax.ShapeDtypeStruct((_N, _H), jnp.float32),
        ],
    )(gat, ew2, c2, h, w0, b0, w2, b2, l2w, l2b, lw, lb, l1n)


def _head_body(h_ref, batc_ref, ce_ref, cw1_ref, cb1_ref, cw2_ref, cb2_ref,
               gwm_ref, gwc_ref, gb_ref, fcmw_ref, fcmb_ref, fccw_ref,
               fccb_ref, tw_ref, tb_ref, out_ref):
    molid = lax.broadcasted_iota(jnp.int32, (_B, _N), 0)
    sel = (molid == batc_ref[...]).astype(jnp.float32)
    mol = jnp.dot(sel, h_ref[...], preferred_element_type=jnp.float32)
    c1 = jnp.maximum(
        jnp.dot(ce_ref[...], cw1_ref[...], preferred_element_type=jnp.float32)
        + cb1_ref[...], 0.0)
    clip = jnp.dot(c1, cw2_ref[...], preferred_element_type=jnp.float32) + cb2_ref[...]
    gl = (jnp.dot(mol, gwm_ref[...], preferred_element_type=jnp.float32)
          + jnp.dot(clip, gwc_ref[...], preferred_element_type=jnp.float32)
          + gb_ref[...])
    g = 1.0 / (1.0 + jnp.exp(-gl))
    fused = (g * (jnp.dot(mol, fcmw_ref[...], preferred_element_type=jnp.float32)
                  + fcmb_ref[...])
             + (1.0 - g) * (jnp.dot(clip, fccw_ref[...],
                                    preferred_element_type=jnp.float32)
                            + fccb_ref[...]))
    out_ref[...] = jnp.dot(fused, tw_ref[...],
                           preferred_element_type=jnp.float32) + tb_ref[...]


def _head(h, batc, ce, cw1, cb1, cw2, cb2, gwm, gwc, gb,
          fcmw, fcmb, fccw, fccb, twp, tbp):
    full = lambda a, b: pl.BlockSpec((a, b), lambda: (0, 0))
    return pl.pallas_call(
        _head_body,
        in_specs=[
            full(_N, _H), full(1, _N), full(_B, 768), full(768, _H),
            full(1, _H), full(_H, _CE), full(1, _CE), full(_H, _H),
            full(_CE, _H), full(1, _H), full(_H, _H), full(1, _H),
            full(_CE, _H), full(1, _H), full(_H, _H), full(1, _H),
        ],
        out_specs=full(_B, _H),
        out_shape=jax.ShapeDtypeStruct((_B, _H), jnp.float32),
    )(h, batc, ce, cw1, cb1, cw2, cb2, gwm, gwc, gb,
      fcmw, fcmb, fccw, fccb, twp, tbp)


def kernel(z, pos, batch, clip_embeddings, emb, mlp_w0, mlp_b0, mlp_w2, mlp_b2,
           lin1_w, lin2_w, lin2_b, lin_w, lin_b, clip_w1, clip_b1, clip_w2,
           clip_b2, gate_w, gate_b, fcm_w, fcm_b, fcc_w, fcc_b, tgt_w, tgt_b):
    z = z.astype(jnp.int32)
    batch = batch.astype(jnp.int32)

    pos_pad = jnp.zeros((_N, 8), jnp.float32).at[:, :3].set(pos)
    posT = pos_pad.T
    batr = batch.reshape(_N, 1)
    batc = batch.reshape(1, _N)

    idx, ew, cc = _build_graph(pos_pad, posT, batr, batc)
    src = idx.reshape(_E)
    ew2 = ew.reshape(_E, 1)
    c2 = cc.reshape(_E, 1)

    emb_pad = jnp.zeros((128, _H), jnp.float32).at[:100, :].set(emb)
    h, x1 = _embed(z.reshape(_N, 1), emb_pad, lin1_w[0])

    w0p = jnp.zeros((_L, _GP, _NF), jnp.float32).at[:, :_G, :].set(mlp_w0)
    for i in range(_L):
        gat = _gather_rows(x1, src)
        h, x1 = _layer(
            gat, ew2, c2, h,
            w0p[i], mlp_b0[i].reshape(1, _NF),
            mlp_w2[i], mlp_b2[i].reshape(1, _NF),
            lin2_w[i], lin2_b[i].reshape(1, _H),
            lin_w[i], lin_b[i].reshape(1, _H),
            lin1_w[(i + 1) % _L],
        )

    twp = jnp.zeros((_H, _H), jnp.float32).at[:, :1].set(tgt_w)
    tbp = jnp.zeros((1, _H), jnp.float32).at[0, 0].set(tgt_b[0])
    out = _head(
        h, batc, clip_embeddings,
        clip_w1, clip_b1.reshape(1, _H),
        clip_w2, clip_b2.reshape(1, _CE),
        gate_w[:_H], gate_w[_H:], gate_b.reshape(1, _H),
        fcm_w, fcm_b.reshape(1, _H),
        fcc_w, fcc_b.reshape(1, _H),
        twp, tbp,
    )
    return out[:, 0:1]


# speedup vs baseline: 1.3406x; 1.3406x over previous
import functools
import math

import jax
import jax.numpy as jnp
from jax import lax
from jax.experimental import pallas as pl
from jax.experimental.pallas import tpu as pltpu
from jax.experimental.pallas import tpu_sc as plsc

_N = 4096
_B = 128
_H = 128
_NF = 128
_G = 50
_GP = 64
_L = 6
_CE = 32
_CUTOFF = 10.0
_K = 32
_E = _N * _K

_STEP = _CUTOFF / (_G - 1)
_COEFF = -0.5 / (_STEP * _STEP)
_LOG2 = math.log(2.0)

_ROWS = 128
_TN = 128
_TE = _TN * _K

_NC = 2
_NS = 16
_NW = _NC * _NS
_EPW = _E // _NW
_CH = 128
_NCH = _EPW // _CH


def _ssp(x):
    m = jnp.maximum(x, 0.0)
    return m + jnp.log(jnp.exp(x - m) + jnp.exp(-m)) - _LOG2


def _graph_body(posr_ref, posc_ref, batr_ref, batc_ref, idx_ref, ew_ref, c_ref):
    t = pl.program_id(0)
    pr = posr_ref[...]
    pc = posc_ref[...]
    sqr = jnp.sum(pr * pr, axis=1, keepdims=True)
    sqc = jnp.sum(pc * pc, axis=0, keepdims=True)
    d2 = sqr + sqc - 2.0 * jnp.dot(pr, pc, preferred_element_type=jnp.float32)
    dist = jnp.sqrt(jnp.maximum(d2, 1e-12))
    colid = lax.broadcasted_iota(jnp.int32, (_ROWS, _N), 1)
    rowid = lax.broadcasted_iota(jnp.int32, (_ROWS, _N), 0) + t * _ROWS
    same = batr_ref[...] == batc_ref[...]
    mask = same & (rowid != colid) & (dist < _CUTOFF)
    rank = jnp.where(mask, dist, 1e9)
    for k in range(_K):
        m = jnp.min(rank, axis=1, keepdims=True)
        amin = jnp.min(jnp.where(rank == m, colid, _N), axis=1, keepdims=True)
        valid = m < _CUTOFF
        idx_ref[:, k:k + 1] = amin
        ew_ref[:, k:k + 1] = jnp.where(valid, m, _CUTOFF)
        c_ref[:, k:k + 1] = jnp.where(
            valid, 0.5 * (jnp.cos(m * (math.pi / _CUTOFF)) + 1.0), 0.0)
        rank = jnp.where(colid == amin, 1e9, rank)


def _build_graph(pos_pad, posT, batr, batc):
    return pl.pallas_call(
        _graph_body,
        grid=(_N // _ROWS,),
        in_specs=[
            pl.BlockSpec((_ROWS, 8), lambda i: (i, 0)),
            pl.BlockSpec((8, _N), lambda i: (0, 0)),
            pl.BlockSpec((_ROWS, 1), lambda i: (i, 0)),
            pl.BlockSpec((1, _N), lambda i: (0, 0)),
        ],
        out_specs=[
            pl.BlockSpec((_ROWS, _K), lambda i: (i, 0)),
            pl.BlockSpec((_ROWS, _K), lambda i: (i, 0)),
            pl.BlockSpec((_ROWS, _K), lambda i: (i, 0)),
        ],
        out_shape=[
            jax.ShapeDtypeStruct((_N, _K), jnp.int32),
            jax.ShapeDtypeStruct((_N, _K), jnp.float32),
            jax.ShapeDtypeStruct((_N, _K), jnp.float32),
        ],
    )(pos_pad, posT, batr, batc)


def _embed_body(z_ref, emb_ref, l1_ref, h_ref, x1_ref):
    oh = (z_ref[...] == lax.broadcasted_iota(jnp.int32, (1, 128), 1))
    h = jnp.dot(oh.astype(jnp.float32), emb_ref[...],
                preferred_element_type=jnp.float32)
    h_ref[...] = h
    x1_ref[...] = jnp.dot(h, l1_ref[...], preferred_element_type=jnp.float32)


def _embed(zcol, emb_pad, lin1_0):
    return pl.pallas_call(
        _embed_body,
        grid=(_N // 256,),
        in_specs=[
            pl.BlockSpec((256, 1), lambda i: (i, 0)),
            pl.BlockSpec((128, _H), lambda i: (0, 0)),
            pl.BlockSpec((_H, _H), lambda i: (0, 0)),
        ],
        out_specs=[
            pl.BlockSpec((256, _H), lambda i: (i, 0)),
            pl.BlockSpec((256, _H), lambda i: (i, 0)),
        ],
        out_shape=[
            jax.ShapeDtypeStruct((_N, _H), jnp.float32),
            jax.ShapeDtypeStruct((_N, _H), jnp.float32),
        ],
    )(zcol, emb_pad, lin1_0)


_CHG = 256
_NG = _EPW // _CHG


def _gather_sc_body(x1_hbm, src_hbm, out_hbm, idx_v, bufa, bufb, xs,
                    sga, sgb, swa, swb):
    if True:
        c = lax.axis_index("c")
        s = lax.axis_index("s")
        wid = s * _NC + c
        base = pl.multiple_of(wid * _EPW, 8)
        rs = pl.multiple_of(s * (_N // _NS), 8)
        pltpu.sync_copy(x1_hbm.at[pl.ds(rs, _N // _NS)],
                        xs.at[pl.ds(rs, _N // _NS)])
        pltpu.sync_copy(src_hbm.at[pl.ds(base, _EPW)], idx_v)
        plsc.subcore_barrier()

        def startg(buf, sem, g):
            o0 = pl.multiple_of(g * _CHG, 8)
            o1 = pl.multiple_of(g * _CHG + _CH, 8)
            pltpu.async_copy(xs.at[idx_v.at[pl.ds(o0, _CH)]],
                             buf.at[pl.ds(0, _CH)], sem)
            pltpu.async_copy(xs.at[idx_v.at[pl.ds(o1, _CH)]],
                             buf.at[pl.ds(_CH, _CH)], sem)

        def waitg(buf, sem):
            pltpu.make_async_copy(xs.at[pl.ds(0, _CHG)], buf, sem).wait()

        def startw(buf, sem, g):
            o = pl.multiple_of(base + g * _CHG, 8)
            pltpu.async_copy(buf, out_hbm.at[pl.ds(o, _CHG)], sem)

        def waitw(buf, sem):
            pltpu.make_async_copy(buf, out_hbm.at[pl.ds(0, _CHG)], sem).wait()

        startg(bufa, sga, 0)

        def body(tt, carry):
            g0 = tt * 2

            @pl.when(tt > 0)
            def _():
                waitw(bufb, swb)
            startg(bufb, sgb, g0 + 1)
            waitg(bufa, sga)
            startw(bufa, swa, g0)

            @pl.when(tt + 1 < _NG // 2)
            def _():
                waitw(bufa, swa)
                startg(bufa, sga, g0 + 2)
            waitg(bufb, sgb)
            startw(bufb, swb, g0 + 1)
            return carry

        lax.fori_loop(0, _NG // 2, body, 0)
        waitw(bufa, swa)
        waitw(bufb, swb)


@functools.lru_cache(maxsize=1)
def _gather_kernel():
    return pl.kernel(
        _gather_sc_body,
        out_type=jax.ShapeDtypeStruct((_E, _H), jnp.float32),
        mesh=plsc.VectorSubcoreMesh(core_axis_name="c", subcore_axis_name="s"),
        scratch_types=[
            pltpu.VMEM((_EPW,), jnp.int32),
            pltpu.VMEM((_CHG, _H), jnp.float32),
            pltpu.VMEM((_CHG, _H), jnp.float32),
            pltpu.VMEM_SHARED((_N, _H), jnp.float32),
            pltpu.SemaphoreType.DMA,
            pltpu.SemaphoreType.DMA,
            pltpu.SemaphoreType.DMA,
            pltpu.SemaphoreType.DMA,
        ],
    )


def _gather_rows(x1, src):
    return _gather_kernel()(x1, src)


def _layer_body(gat_ref, ew_ref, c_ref, h_ref, w0_ref, b0_ref, w2_ref, b2_ref,
                l2w_ref, l2b_ref, lw_ref, lb_ref, l1n_ref, hout_ref, x1out_ref):
    ew3 = ew_ref[...][:, :, None]
    off = (lax.broadcasted_iota(jnp.int32, (1, 1, _GP), 2).astype(jnp.float32)
           * _STEP)
    ea = jnp.exp(_COEFF * (ew3 - off) ** 2).reshape(_TE, _GP)
    f = _ssp(jnp.dot(ea, w0_ref[...], preferred_element_type=jnp.float32)
             + b0_ref[...])
    wf = jnp.dot(f, w2_ref[...], preferred_element_type=jnp.float32) + b2_ref[...]
    msg3 = (gat_ref[...].reshape(_TN, _K, _H) * wf.reshape(_TN, _K, _H)
            * c_ref[...][:, :, None])
    agg = jnp.sum(msg3, axis=1)
    x2 = _ssp(jnp.dot(agg, l2w_ref[...], preferred_element_type=jnp.float32)
              + l2b_ref[...])
    x2 = jnp.dot(x2, lw_ref[...], preferred_element_type=jnp.float32) + lb_ref[...]
    h = h_ref[...] + x2
    hout_ref[...] = h
    x1out_ref[...] = jnp.dot(h, l1n_ref[...], preferred_element_type=jnp.float32)


def _layer(gat, ew2, c2, h, w0, b0, w2, b2, l2w, l2b, lw, lb, l1n):
    full = lambda a, b: pl.BlockSpec((a, b), lambda i: (0, 0))
    return pl.pallas_call(
        _layer_body,
        grid=(_N // _TN,),
        in_specs=[
            pl.BlockSpec((_TE, _H), lambda i: (i, 0)),
            pl.BlockSpec((_TN, _K), lambda i: (i, 0)),
            pl.BlockSpec((_TN, _K), lambda i: (i, 0)),
            pl.BlockSpec((_TN, _H), lambda i: (i, 0)),
            full(_GP, _NF), full(1, _NF), full(_NF, _NF), full(1, _NF),
            full(_NF, _H), full(1, _H), full(_H, _H), full(1, _H),
            full(_H, _H),
        ],
        out_specs=[
            pl.BlockSpec((_TN, _H), lambda i: (i, 0)),
            pl.BlockSpec((_TN, _H), lambda i: (i, 0)),
        ],
        out_shape=[
            jax.ShapeDtypeStruct((_N, _H), jnp.float32),
            jax.ShapeDtypeStruct((_N, _H), jnp.float32),
        ],
    )(gat, ew2, c2, h, w0, b0, w2, b2, l2w, l2b, lw, lb, l1n)


def _head_body(h_ref, batc_ref, ce_ref, cw1_ref, cb1_ref, cw2_ref, cb2_ref,
               gwm_ref, gwc_ref, gb_ref, fcmw_ref, fcmb_ref, fccw_ref,
               fccb_ref, tw_ref, tb_ref, out_ref):
    molid = lax.broadcasted_iota(jnp.int32, (_B, _N), 0)
    sel = (molid == batc_ref[...]).astype(jnp.float32)
    mol = jnp.dot(sel, h_ref[...], preferred_element_type=jnp.float32)
    c1 = jnp.maximum(
        jnp.dot(ce_ref[...], cw1_ref[...], preferred_element_type=jnp.float32)
        + cb1_ref[...], 0.0)
    clip = jnp.dot(c1, cw2_ref[...], preferred_element_type=jnp.float32) + cb2_ref[...]
    gl = (jnp.dot(mol, gwm_ref[...], preferred_element_type=jnp.float32)
          + jnp.dot(clip, gwc_ref[...], preferred_element_type=jnp.float32)
          + gb_ref[...])
    g = 1.0 / (1.0 + jnp.exp(-gl))
    fused = (g * (jnp.dot(mol, fcmw_ref[...], preferred_element_type=jnp.float32)
                  + fcmb_ref[...])
             + (1.0 - g) * (jnp.dot(clip, fccw_ref[...],
                                    preferred_element_type=jnp.float32)
                            + fccb_ref[...]))
    out_ref[...] = jnp.dot(fused, tw_ref[...],
                           preferred_element_type=jnp.float32) + tb_ref[...]


def _head(h, batc, ce, cw1, cb1, cw2, cb2, gwm, gwc, gb,
          fcmw, fcmb, fccw, fccb, twp, tbp):
    full = lambda a, b: pl.BlockSpec((a, b), lambda: (0, 0))
    return pl.pallas_call(
        _head_body,
        in_specs=[
            full(_N, _H), full(1, _N), full(_B, 768), full(768, _H),
            full(1, _H), full(_H, _CE), full(1, _CE), full(_H, _H),
            full(_CE, _H), full(1, _H), full(_H, _H), full(1, _H),
            full(_CE, _H), full(1, _H), full(_H, _H), full(1, _H),
        ],
        out_specs=full(_B, _H),
        out_shape=jax.ShapeDtypeStruct((_B, _H), jnp.float32),
    )(h, batc, ce, cw1, cb1, cw2, cb2, gwm, gwc, gb,
      fcmw, fcmb, fccw, fccb, twp, tbp)


def kernel(z, pos, batch, clip_embeddings, emb, mlp_w0, mlp_b0, mlp_w2, mlp_b2,
           lin1_w, lin2_w, lin2_b, lin_w, lin_b, clip_w1, clip_b1, clip_w2,
           clip_b2, gate_w, gate_b, fcm_w, fcm_b, fcc_w, fcc_b, tgt_w, tgt_b):
    z = z.astype(jnp.int32)
    batch = batch.astype(jnp.int32)

    pos_pad = jnp.zeros((_N, 8), jnp.float32).at[:, :3].set(pos)
    posT = pos_pad.T
    batr = batch.reshape(_N, 1)
    batc = batch.reshape(1, _N)

    idx, ew, cc = _build_graph(pos_pad, posT, batr, batc)
    src = idx.reshape(_E)

    emb_pad = jnp.zeros((128, _H), jnp.float32).at[:100, :].set(emb)
    h, x1 = _embed(z.reshape(_N, 1), emb_pad, lin1_w[0])

    w0p = jnp.zeros((_L, _GP, _NF), jnp.float32).at[:, :_G, :].set(mlp_w0)
    for i in range(_L):
        gat = _gather_rows(x1, src)
        h, x1 = _layer(
            gat, ew, cc, h,
            w0p[i], mlp_b0[i].reshape(1, _NF),
            mlp_w2[i], mlp_b2[i].reshape(1, _NF),
            lin2_w[i], lin2_b[i].reshape(1, _H),
            lin_w[i], lin_b[i].reshape(1, _H),
            lin1_w[(i + 1) % _L],
        )

    twp = jnp.zeros((_H, _H), jnp.float32).at[:, :1].set(tgt_w)
    tbp = jnp.zeros((1, _H), jnp.float32).at[0, 0].set(tgt_b[0])
    out = _head(
        h, batc, clip_embeddings,
        clip_w1, clip_b1.reshape(1, _H),
        clip_w2, clip_b2.reshape(1, _CE),
        gate_w[:_H], gate_w[_H:], gate_b.reshape(1, _H),
        fcm_w, fcm_b.reshape(1, _H),
        fcc_w, fcc_b.reshape(1, _H),
        twp, tbp,
    )
    return out[:, 0:1]


# speedup vs baseline: 1.3413x; 1.0005x over previous
import functools
import math

import jax
import jax.numpy as jnp
from jax import lax
from jax.experimental import pallas as pl
from jax.experimental.pallas import tpu as pltpu
from jax.experimental.pallas import tpu_sc as plsc

_N = 4096
_B = 128
_H = 128
_NF = 128
_G = 50
_GP = 64
_L = 6
_CE = 32
_CUTOFF = 10.0
_K = 32
_E = _N * _K

_STEP = _CUTOFF / (_G - 1)
_COEFF = -0.5 / (_STEP * _STEP)
_LOG2 = math.log(2.0)

_ROWS = 128
_TN = 128
_TE = _TN * _K

_NC = 2
_NS = 16
_NW = _NC * _NS
_EPW = _E // _NW
_CH = 128


def _ssp(x):
    m = jnp.maximum(x, 0.0)
    return m + jnp.log(jnp.exp(x - m) + jnp.exp(-m)) - _LOG2


def _graph_body(posr_ref, posc_ref, batr_ref, batc_ref, idx_ref, ew_ref, c_ref):
    t = pl.program_id(0)
    pr = posr_ref[...]
    pc = posc_ref[...]
    sqr = jnp.sum(pr * pr, axis=1, keepdims=True)
    sqc = jnp.sum(pc * pc, axis=0, keepdims=True)
    d2 = sqr + sqc - 2.0 * jnp.dot(pr, pc, preferred_element_type=jnp.float32)
    dist = jnp.sqrt(jnp.maximum(d2, 1e-12))
    colid = lax.broadcasted_iota(jnp.int32, (_ROWS, _N), 1)
    rowid = lax.broadcasted_iota(jnp.int32, (_ROWS, _N), 0) + t * _ROWS
    same = batr_ref[...] == batc_ref[...]
    mask = same & (rowid != colid) & (dist < _CUTOFF)
    rank = jnp.where(mask, dist, 1e9)
    for k in range(_K):
        m = jnp.min(rank, axis=1, keepdims=True)
        amin = jnp.min(jnp.where(rank == m, colid, _N), axis=1, keepdims=True)
        valid = m < _CUTOFF
        idx_ref[:, k:k + 1] = amin
        ew_ref[:, k:k + 1] = jnp.where(valid, m, _CUTOFF)
        c_ref[:, k:k + 1] = jnp.where(
            valid, 0.5 * (jnp.cos(m * (math.pi / _CUTOFF)) + 1.0), 0.0)
        rank = jnp.where(colid == amin, 1e9, rank)


def _build_graph(pos_pad, posT, batr, batc):
    return pl.pallas_call(
        _graph_body,
        grid=(_N // _ROWS,),
        in_specs=[
            pl.BlockSpec((_ROWS, 8), lambda i: (i, 0)),
            pl.BlockSpec((8, _N), lambda i: (0, 0)),
            pl.BlockSpec((_ROWS, 1), lambda i: (i, 0)),
            pl.BlockSpec((1, _N), lambda i: (0, 0)),
        ],
        out_specs=[
            pl.BlockSpec((_ROWS, _K), lambda i: (i, 0)),
            pl.BlockSpec((_ROWS, _K), lambda i: (i, 0)),
            pl.BlockSpec((_ROWS, _K), lambda i: (i, 0)),
        ],
        out_shape=[
            jax.ShapeDtypeStruct((_N, _K), jnp.int32),
            jax.ShapeDtypeStruct((_N, _K), jnp.float32),
            jax.ShapeDtypeStruct((_N, _K), jnp.float32),
        ],
    )(pos_pad, posT, batr, batc)


def _embed_body(z_ref, emb_ref, l1_ref, h_ref, x1_ref):
    oh = (z_ref[...] == lax.broadcasted_iota(jnp.int32, (1, 128), 1))
    h = jnp.dot(oh.astype(jnp.float32), emb_ref[...],
                preferred_element_type=jnp.float32)
    h_ref[...] = h
    x1_ref[...] = jnp.dot(h, l1_ref[...], preferred_element_type=jnp.float32)


def _embed(zcol, emb_pad, lin1_0):
    return pl.pallas_call(
        _embed_body,
        grid=(_N // 256,),
        in_specs=[
            pl.BlockSpec((256, 1), lambda i: (i, 0)),
            pl.BlockSpec((128, _H), lambda i: (0, 0)),
            pl.BlockSpec((_H, _H), lambda i: (0, 0)),
        ],
        out_specs=[
            pl.BlockSpec((256, _H), lambda i: (i, 0)),
            pl.BlockSpec((256, _H), lambda i: (i, 0)),
        ],
        out_shape=[
            jax.ShapeDtypeStruct((_N, _H), jnp.float32),
            jax.ShapeDtypeStruct((_N, _H), jnp.float32),
        ],
    )(zcol, emb_pad, lin1_0)


_CHG = 256
_NG = _EPW // _CHG


def _gather_sc_body(x1_hbm, src_hbm, out_hbm, idx_v, bufa, bufb, xs,
                    sga, sgb, swa, swb):
    c = lax.axis_index("c")
    s = lax.axis_index("s")
    wid = s * _NC + c
    base = pl.multiple_of(wid * _EPW, 8)
    rs = pl.multiple_of(s * (_N // _NS), 8)
    pltpu.sync_copy(x1_hbm.at[pl.ds(rs, _N // _NS)],
                    xs.at[pl.ds(rs, _N // _NS)])
    pltpu.sync_copy(src_hbm.at[pl.ds(base, _EPW)], idx_v)
    plsc.subcore_barrier()

    def startg(buf, sem, g):
        o0 = pl.multiple_of(g * _CHG, 8)
        o1 = pl.multiple_of(g * _CHG + _CH, 8)
        pltpu.async_copy(xs.at[idx_v.at[pl.ds(o0, _CH)]],
                         buf.at[pl.ds(0, _CH)], sem)
        pltpu.async_copy(xs.at[idx_v.at[pl.ds(o1, _CH)]],
                         buf.at[pl.ds(_CH, _CH)], sem)

    def waitg(buf, sem):
        pltpu.make_async_copy(xs.at[pl.ds(0, _CHG)], buf, sem).wait()

    def startw(buf, sem, g):
        o = pl.multiple_of(base + g * _CHG, 8)
        pltpu.async_copy(buf, out_hbm.at[pl.ds(o, _CHG)], sem)

    def waitw(buf, sem):
        pltpu.make_async_copy(buf, out_hbm.at[pl.ds(0, _CHG)], sem).wait()

    startg(bufa, sga, 0)

    def body(tt, carry):
        g0 = tt * 2

        @pl.when(tt > 0)
        def _():
            waitw(bufb, swb)
        startg(bufb, sgb, g0 + 1)
        waitg(bufa, sga)
        startw(bufa, swa, g0)

        @pl.when(tt + 1 < _NG // 2)
        def _():
            waitw(bufa, swa)
            startg(bufa, sga, g0 + 2)
        waitg(bufb, sgb)
        startw(bufb, swb, g0 + 1)
        return carry

    lax.fori_loop(0, _NG // 2, body, 0)
    waitw(bufa, swa)
    waitw(bufb, swb)


@functools.lru_cache(maxsize=1)
def _gather_kernel():
    return pl.kernel(
        _gather_sc_body,
        out_type=jax.ShapeDtypeStruct((_E, _H), jnp.float32),
        mesh=plsc.VectorSubcoreMesh(core_axis_name="c", subcore_axis_name="s"),
        scratch_types=[
            pltpu.VMEM((_EPW,), jnp.int32),
            pltpu.VMEM((_CHG, _H), jnp.float32),
            pltpu.VMEM((_CHG, _H), jnp.float32),
            pltpu.VMEM_SHARED((_N, _H), jnp.float32),
            pltpu.SemaphoreType.DMA,
            pltpu.SemaphoreType.DMA,
            pltpu.SemaphoreType.DMA,
            pltpu.SemaphoreType.DMA,
        ],
    )


def _gather_rows(x1, src):
    return _gather_kernel()(x1, src)


def _layer_body(gat_ref, ew_ref, c_ref, h_ref, w0_ref, b0_ref, w2_ref, b2_ref,
                l2w_ref, l2b_ref, lw_ref, lb_ref, l1n_ref, hout_ref, x1out_ref):
    ew3 = ew_ref[...][:, :, None]
    off = (lax.broadcasted_iota(jnp.int32, (1, 1, _GP), 2).astype(jnp.float32)
           * _STEP)
    ea = jnp.exp(_COEFF * (ew3 - off) ** 2).reshape(_TE, _GP)
    f = _ssp(jnp.dot(ea, w0_ref[...], preferred_element_type=jnp.float32)
             + b0_ref[...])
    wf = jnp.dot(f, w2_ref[...], preferred_element_type=jnp.float32) + b2_ref[...]
    msg3 = (gat_ref[...].reshape(_TN, _K, _H) * wf.reshape(_TN, _K, _H)
            * c_ref[...][:, :, None])
    agg = jnp.sum(msg3, axis=1)
    x2 = _ssp(jnp.dot(agg, l2w_ref[...], preferred_element_type=jnp.float32)
              + l2b_ref[...])
    x2 = jnp.dot(x2, lw_ref[...], preferred_element_type=jnp.float32) + lb_ref[...]
    h = h_ref[...] + x2
    hout_ref[...] = h
    x1out_ref[...] = jnp.dot(h, l1n_ref[...], preferred_element_type=jnp.float32)


def _layer(gat, ew2, c2, h, w0, b0, w2, b2, l2w, l2b, lw, lb, l1n):
    full = lambda a, b: pl.BlockSpec((a, b), lambda i: (0, 0))
    return pl.pallas_call(
        _layer_body,
        grid=(_N // _TN,),
        in_specs=[
            pl.BlockSpec((_TE, _H), lambda i: (i, 0)),
            pl.BlockSpec((_TN, _K), lambda i: (i, 0)),
            pl.BlockSpec((_TN, _K), lambda i: (i, 0)),
            pl.BlockSpec((_TN, _H), lambda i: (i, 0)),
            full(_GP, _NF), full(1, _NF), full(_NF, _NF), full(1, _NF),
            full(_NF, _H), full(1, _H), full(_H, _H), full(1, _H),
            full(_H, _H),
        ],
        out_specs=[
            pl.BlockSpec((_TN, _H), lambda i: (i, 0)),
            pl.BlockSpec((_TN, _H), lambda i: (i, 0)),
        ],
        out_shape=[
            jax.ShapeDtypeStruct((_N, _H), jnp.float32),
            jax.ShapeDtypeStruct((_N, _H), jnp.float32),
        ],
    )(gat, ew2, c2, h, w0, b0, w2, b2, l2w, l2b, lw, lb, l1n)


def _head_body(h_ref, batc_ref, ce_ref, cw1_ref, cb1_ref, cw2_ref, cb2_ref,
               gwm_ref, gwc_ref, gb_ref, fcmw_ref, fcmb_ref, fccw_ref,
               fccb_ref, tw_ref, tb_ref, out_ref):
    molid = lax.broadcasted_iota(jnp.int32, (_B, _N), 0)
    sel = (molid == batc_ref[...]).astype(jnp.float32)
    mol = jnp.dot(sel, h_ref[...], preferred_element_type=jnp.float32)
    c1 = jnp.maximum(
        jnp.dot(ce_ref[...], cw1_ref[...], preferred_element_type=jnp.float32)
        + cb1_ref[...], 0.0)
    clip = jnp.dot(c1, cw2_ref[...], preferred_element_type=jnp.float32) + cb2_ref[...]
    gl = (jnp.dot(mol, gwm_ref[...], preferred_element_type=jnp.float32)
          + jnp.dot(clip, gwc_ref[...], preferred_element_type=jnp.float32)
          + gb_ref[...])
    g = 1.0 / (1.0 + jnp.exp(-gl))
    fused = (g * (jnp.dot(mol, fcmw_ref[...], preferred_element_type=jnp.float32)
                  + fcmb_ref[...])
             + (1.0 - g) * (jnp.dot(clip, fccw_ref[...],
                                    preferred_element_type=jnp.float32)
                            + fccb_ref[...]))
    out_ref[...] = jnp.dot(fused, tw_ref[...],
                           preferred_element_type=jnp.float32) + tb_ref[...]


def _head(h, batc, ce, cw1, cb1, cw2, cb2, gwm, gwc, gb,
          fcmw, fcmb, fccw, fccb, twp, tbp):
    full = lambda a, b: pl.BlockSpec((a, b), lambda: (0, 0))
    return pl.pallas_call(
        _head_body,
        in_specs=[
            full(_N, _H), full(1, _N), full(_B, 768), full(768, _H),
            full(1, _H), full(_H, _CE), full(1, _CE), full(_H, _H),
            full(_CE, _H), full(1, _H), full(_H, _H), full(1, _H),
            full(_CE, _H), full(1, _H), full(_H, _H), full(1, _H),
        ],
        out_specs=full(_B, _H),
        out_shape=jax.ShapeDtypeStruct((_B, _H), jnp.float32),
    )(h, batc, ce, cw1, cb1, cw2, cb2, gwm, gwc, gb,
      fcmw, fcmb, fccw, fccb, twp, tbp)


def kernel(z, pos, batch, clip_embeddings, emb, mlp_w0, mlp_b0, mlp_w2, mlp_b2,
           lin1_w, lin2_w, lin2_b, lin_w, lin_b, clip_w1, clip_b1, clip_w2,
           clip_b2, gate_w, gate_b, fcm_w, fcm_b, fcc_w, fcc_b, tgt_w, tgt_b):
    z = z.astype(jnp.int32)
    batch = batch.astype(jnp.int32)

    pos_pad = jnp.zeros((_N, 8), jnp.float32).at[:, :3].set(pos)
    posT = pos_pad.T
    batr = batch.reshape(_N, 1)
    batc = batch.reshape(1, _N)

    idx, ew, cc = _build_graph(pos_pad, posT, batr, batc)
    src = idx.reshape(_E)

    emb_pad = jnp.zeros((128, _H), jnp.float32).at[:100, :].set(emb)
    h, x1 = _embed(z.reshape(_N, 1), emb_pad, lin1_w[0])

    w0p = jnp.zeros((_L, _GP, _NF), jnp.float32).at[:, :_G, :].set(mlp_w0)
    for i in range(_L):
        gat = _gather_rows(x1, src)
        h, x1 = _layer(
            gat, ew, cc, h,
            w0p[i], mlp_b0[i].reshape(1, _NF),
            mlp_w2[i], mlp_b2[i].reshape(1, _NF),
            lin2_w[i], lin2_b[i].reshape(1, _H),
            lin_w[i], lin_b[i].reshape(1, _H),
            lin1_w[(i + 1) % _L],
        )

    twp = jnp.zeros((_H, _H), jnp.float32).at[:, :1].set(tgt_w)
    tbp = jnp.zeros((1, _H), jnp.float32).at[0, 0].set(tgt_b[0])
    out = _head(
        h, batc, clip_embeddings,
        clip_w1, clip_b1.reshape(1, _H),
        clip_w2, clip_b2.reshape(1, _CE),
        gate_w[:_H], gate_w[_H:], gate_b.reshape(1, _H),
        fcm_w, fcm_b.reshape(1, _H),
        fcc_w, fcc_b.reshape(1, _H),
        twp, tbp,
    )
    return out[:, 0:1]
